# baseline (speedup 1.0000x reference)
import jax
import jax.numpy as jnp
from jax import lax
from jax.experimental import pallas as pl
from jax.experimental.pallas import tpu as pltpu
from jax.experimental.pallas import tpu_sc as plsc

_B = 16384
_MF = 16
_E = 32
_NC = 2
_NS = 16
_NW = _NC * _NS
_BPW = _B // _NW


def _sc_gather_body(u_hbm, i_hbm, mfu_t, mfi_t, mlu_t, mli_t,
                    mfu_o, mfi_o, mlu_o, mli_o,
                    uidx, iidx, mfu_v, mfi_v, mlu_v, mli_v, sem):
    wid = lax.axis_index("s") * _NC + lax.axis_index("c")
    base = wid * _BPW
    pltpu.sync_copy(u_hbm.at[pl.ds(base, _BPW)], uidx)
    pltpu.sync_copy(i_hbm.at[pl.ds(base, _BPW)], iidx)
    c1 = pltpu.async_copy(mfu_t.at[uidx], mfu_v, sem)
    c2 = pltpu.async_copy(mfi_t.at[iidx], mfi_v, sem)
    c3 = pltpu.async_copy(mlu_t.at[uidx], mlu_v, sem)
    c4 = pltpu.async_copy(mli_t.at[iidx], mli_v, sem)
    c1.wait()
    c2.wait()
    c3.wait()
    c4.wait()
    pltpu.sync_copy(mfu_v, mfu_o.at[pl.ds(base, _BPW)])
    pltpu.sync_copy(mfi_v, mfi_o.at[pl.ds(base, _BPW)])
    pltpu.sync_copy(mlu_v, mlu_o.at[pl.ds(base, _BPW)])
    pltpu.sync_copy(mli_v, mli_o.at[pl.ds(base, _BPW)])


_sc_gather = pl.kernel(
    _sc_gather_body,
    out_type=[
        jax.ShapeDtypeStruct((_B, _MF), jnp.float32),
        jax.ShapeDtypeStruct((_B, _MF), jnp.float32),
        jax.ShapeDtypeStruct((_B, _E), jnp.float32),
        jax.ShapeDtypeStruct((_B, _E), jnp.float32),
    ],
    mesh=plsc.VectorSubcoreMesh(core_axis_name="c", subcore_axis_name="s"),
    compiler_params=pltpu.CompilerParams(use_tc_tiling_on_sc=False),
    scratch_types=[
        pltpu.VMEM((_BPW,), jnp.int32),
        pltpu.VMEM((_BPW,), jnp.int32),
        pltpu.VMEM((_BPW, _MF), jnp.float32),
        pltpu.VMEM((_BPW, _MF), jnp.float32),
        pltpu.VMEM((_BPW, _E), jnp.float32),
        pltpu.VMEM((_BPW, _E), jnp.float32),
        pltpu.SemaphoreType.DMA,
    ],
)


_BLK = 2048


def _tc_dense_body(mfu, mfi, mlu, mli, w1a, w1b, b1, w2, b2, wp, bp, out):
    h1 = jnp.dot(mlu[...], w1a[...], preferred_element_type=jnp.float32)
    h1 = h1 + jnp.dot(mli[...], w1b[...], preferred_element_type=jnp.float32)
    h1 = jnp.maximum(h1 + b1[...], 0.0)
    h2 = jnp.dot(h1, w2[...], preferred_element_type=jnp.float32)
    h2 = jnp.maximum(h2 + b2[...], 0.0)
    mf = mfu[...] * mfi[...]
    pv = jnp.concatenate([mf * 0.5, h2 * 0.5], axis=1)
    logit = jnp.sum(pv * wp[...], axis=1, keepdims=True) + bp[...]
    out[...] = 1.0 / (1.0 + jnp.exp(-logit))


def _tc_dense(mfu, mfi, mlu, mli, w1a, w1b, b1, w2, b2, wp, bp):
    n_blk = _B // _BLK
    full = lambda shape: pl.BlockSpec(shape, lambda i: (0, 0))
    return pl.pallas_call(
        _tc_dense_body,
        grid=(n_blk,),
        in_specs=[
            pl.BlockSpec((_BLK, _MF), lambda i: (i, 0)),
            pl.BlockSpec((_BLK, _MF), lambda i: (i, 0)),
            pl.BlockSpec((_BLK, _E), lambda i: (i, 0)),
            pl.BlockSpec((_BLK, _E), lambda i: (i, 0)),
            full((_E, 32)),
            full((_E, 32)),
            full((1, 32)),
            full((32, 16)),
            full((1, 16)),
            full((1, 32)),
            full((1, 1)),
        ],
        out_specs=pl.BlockSpec((_BLK, 1), lambda i: (i, 0)),
        out_shape=jax.ShapeDtypeStruct((_B, 1), jnp.float32),
    )(mfu, mfi, mlu, mli, w1a, w1b, b1, w2, b2, wp, bp)


def kernel(user_input, item_input, mf_user_table, mf_item_table,
           mlp_user_table, mlp_item_table, W1, b1, W2, b2, Wp, bp):
    u = user_input.reshape(-1)
    i = item_input.reshape(-1)
    mfu, mfi, mlu, mli = _sc_gather(
        u, i, mf_user_table, mf_item_table, mlp_user_table, mlp_item_table)
    return _tc_dense(
        mfu, mfi, mlu, mli,
        W1[:_E], W1[_E:], b1.reshape(1, 32), W2, b2.reshape(1, 16),
        Wp.reshape(1, _MF + 16), bp.reshape(1, 1))
